# baseline (speedup 1.0000x reference)
import jax
import jax.numpy as jnp
import numpy as np
from jax import lax
from jax.experimental import pallas as pl
from jax.experimental.pallas import tpu as pltpu
from jax.experimental.pallas import tpu_sc as plsc

_LOSS_WEIGHT = 1.0
_B, _C, _H, _W = 8, 3, 512, 512
_N = _H * _W
_PAIRS = 1000
_WPB = 4
_PPW = _PAIRS // _WPB
_PPW_PAD = 256
_NW = 32
_ELEMS_PER_SIDE = _C * _PPW_PAD
_IDX_CHUNK = 128
_N_CHUNKS = _ELEMS_PER_SIDE // _IDX_CHUNK
_FLAT = _B * _C * _N


def _tf2x32(k1, k2, x1, x2):
    rot = ((13, 15, 26, 6), (17, 29, 16, 24))
    ks = (np.uint32(k1), np.uint32(k2),
          np.uint32(np.uint32(k1) ^ np.uint32(k2) ^ np.uint32(0x1BD11BDA)))
    x1 = (x1 + ks[0]).astype(np.uint32)
    x2 = (x2 + ks[1]).astype(np.uint32)
    for i in range(5):
        for r in rot[i % 2]:
            x1 = (x1 + x2).astype(np.uint32)
            x2 = ((x2 << np.uint32(r)) | (x2 >> np.uint32(32 - r))).astype(
                np.uint32)
            x2 = x2 ^ x1
        x1 = (x1 + ks[(i + 1) % 3]).astype(np.uint32)
        x2 = (x2 + ks[(i + 2) % 3] + np.uint32(i + 1)).astype(np.uint32)
    return x1, x2


def _np_randint(key, n, maxval):
    b1, b2 = _tf2x32(key[0], key[1], np.zeros(2, np.uint32),
                     np.arange(2, dtype=np.uint32))
    out = np.empty((2, n), np.uint32)
    for i, sub in enumerate(((b1[0], b2[0]), (b1[1], b2[1]))):
        h, l = _tf2x32(sub[0], sub[1], np.zeros(n, np.uint32),
                       np.arange(n, dtype=np.uint32))
        out[i] = h ^ l
    span = maxval
    mult = (((2 ** 16 % span) * (2 ** 16 % span)) & 0xFFFFFFFF) % span
    off = ((out[0] % span) * np.uint64(mult) + out[1] % span) % span
    return off.astype(np.int32)


def _fold_in(key, data):
    o1, o2 = _tf2x32(key[0], key[1], np.zeros(1, np.uint32),
                     np.full(1, data, np.uint32))
    return (o1[0], o2[0])


def _build_index_tables():
    base = (np.uint32(0), np.uint32(42))
    idx1 = np.stack([_np_randint(_fold_in(base, 2 * b), _PAIRS, _N)
                     for b in range(_B)])
    idx2 = np.stack([_np_randint(_fold_in(base, 2 * b + 1), _PAIRS, _N)
                     for b in range(_B)])
    b_of_w = np.arange(_NW, dtype=np.int32) // _WPB
    chan = np.arange(_C, dtype=np.int32)

    def tables(idx):
        pixw = np.zeros((_NW, _PPW_PAD), np.int32)
        pixw[:, :_PPW] = idx.reshape(_NW, _PPW)
        flat = ((b_of_w[:, None, None] * _C + chan[None, :, None]) * _N
                + pixw[:, None, :])
        return np.ascontiguousarray(
            flat.reshape(_NW, _N_CHUNKS, _IDX_CHUNK).astype(np.int32))

    return tables(idx1), tables(idx2)


_F1, _F2 = _build_index_tables()


def _sqrt16(x):
    i = plsc.bitcast(x, jnp.int32)
    y = plsc.bitcast((i >> 1) + 0x1FBD1DF5, jnp.float32)
    for _ in range(3):
        y = 0.5 * (y + x / y)
    return y


def _sc_body(diff_hbm, ich_hbm, f1_hbm, f2_hbm, out_hbm,
             f1_v, f2_v, gi1, gi2, gd1, gd2, out_v, sem):
    wid = lax.axis_index("s") * 2 + lax.axis_index("c")

    pltpu.sync_copy(f1_hbm.at[wid], f1_v)
    pltpu.sync_copy(f2_hbm.at[wid], f2_v)

    copies = []
    for i in range(_N_CHUNKS):
        dst = pl.ds(i * _IDX_CHUNK, _IDX_CHUNK)
        copies.append(pltpu.async_copy(ich_hbm.at[f1_v.at[i]], gi1.at[dst], sem))
        copies.append(pltpu.async_copy(ich_hbm.at[f2_v.at[i]], gi2.at[dst], sem))
        copies.append(pltpu.async_copy(diff_hbm.at[f1_v.at[i]], gd1.at[dst], sem))
        copies.append(pltpu.async_copy(diff_hbm.at[f2_v.at[i]], gd2.at[dst], sem))
    for cp in copies:
        cp.wait()

    lane_iota = lax.iota(jnp.int32, 16)
    acc_s = jnp.zeros((16,), jnp.float32)
    acc_c = jnp.zeros((16,), jnp.float32)
    for j in range(_PPW_PAD // 16):
        d2_chro = jnp.zeros((16,), jnp.float32)
        d2_diff = jnp.zeros((16,), jnp.float32)
        for c in range(_C):
            sl = pl.ds(c * _PPW_PAD + j * 16, 16)
            d = gi1[sl] - gi2[sl]
            d2_chro = d2_chro + d * d
            e = gd1[sl] - gd2[sl]
            d2_diff = d2_diff + e * e
        same = d2_chro < 0.25
        valid = (j * 16 + lane_iota) < _PPW
        m = jnp.logical_and(same, valid)
        acc_s = acc_s + jnp.where(m, _sqrt16(d2_diff), 0.0)
        acc_c = acc_c + jnp.where(m, 1.0, 0.0)

    s = jnp.sum(acc_s)
    cnt = jnp.sum(acc_c)
    out_v[...] = jnp.where(lane_iota == 0, s,
                           jnp.where(lane_iota == 1, cnt, 0.0))
    pltpu.sync_copy(out_v, out_hbm.at[wid])


def _cluster_loss_impl(diffuse, Ichro):
    one = lax.optimization_barrier(jnp.float32(1.0))
    diff_flat = (diffuse * one).reshape(_FLAT)
    ich_flat = Ichro.reshape(_FLAT)
    mesh = plsc.VectorSubcoreMesh(core_axis_name="c", subcore_axis_name="s",
                                  num_cores=2, num_subcores=16)
    fn = pl.kernel(
        _sc_body,
        out_type=jax.ShapeDtypeStruct((_NW, 16), jnp.float32),
        mesh=mesh,
        scratch_types=[
            pltpu.VMEM((_N_CHUNKS, _IDX_CHUNK), jnp.int32),
            pltpu.VMEM((_N_CHUNKS, _IDX_CHUNK), jnp.int32),
            pltpu.VMEM((_ELEMS_PER_SIDE,), jnp.float32),
            pltpu.VMEM((_ELEMS_PER_SIDE,), jnp.float32),
            pltpu.VMEM((_ELEMS_PER_SIDE,), jnp.float32),
            pltpu.VMEM((_ELEMS_PER_SIDE,), jnp.float32),
            pltpu.VMEM((16,), jnp.float32),
            pltpu.SemaphoreType.DMA,
        ],
        compiler_params=pltpu.CompilerParams(needs_layout_passes=False),
    )
    partials = fn(diff_flat, ich_flat, jnp.asarray(_F1), jnp.asarray(_F2))
    s_b = partials[:, 0].reshape(_B, _WPB).sum(axis=1)
    c_b = partials[:, 1].reshape(_B, _WPB).sum(axis=1)
    loss_b = jnp.where(c_b > 0, s_b / jnp.maximum(c_b, 1.0), 0.0)
    return (_LOSS_WEIGHT * jnp.sum(loss_b) / _B).astype(jnp.float32)


_cluster_loss = jax.jit(_cluster_loss_impl)


def kernel(diffuse, Ichro):
    return _cluster_loss(diffuse, Ichro)


# speedup vs baseline: 1.2112x; 1.2112x over previous
import jax
import jax.numpy as jnp
import numpy as np
from jax import lax
from jax.experimental import pallas as pl
from jax.experimental.pallas import tpu as pltpu
from jax.experimental.pallas import tpu_sc as plsc

_LOSS_WEIGHT = 1.0
_B, _C, _H, _W = 8, 3, 512, 512
_N = _H * _W
_PAIRS = 1000
_WPB = 4
_PPW = _PAIRS // _WPB
_PPW_PAD = 256
_NW = 32
_ELEMS_PER_SIDE = _C * _PPW_PAD
_IDX_CHUNK = 128
_N_CHUNKS = _ELEMS_PER_SIDE // _IDX_CHUNK
_FLAT = _B * _C * _N


def _tf2x32(k1, k2, x1, x2):
    rot = ((13, 15, 26, 6), (17, 29, 16, 24))
    ks = (np.uint32(k1), np.uint32(k2),
          np.uint32(np.uint32(k1) ^ np.uint32(k2) ^ np.uint32(0x1BD11BDA)))
    x1 = (x1 + ks[0]).astype(np.uint32)
    x2 = (x2 + ks[1]).astype(np.uint32)
    for i in range(5):
        for r in rot[i % 2]:
            x1 = (x1 + x2).astype(np.uint32)
            x2 = ((x2 << np.uint32(r)) | (x2 >> np.uint32(32 - r))).astype(
                np.uint32)
            x2 = x2 ^ x1
        x1 = (x1 + ks[(i + 1) % 3]).astype(np.uint32)
        x2 = (x2 + ks[(i + 2) % 3] + np.uint32(i + 1)).astype(np.uint32)
    return x1, x2


def _np_randint(key, n, maxval):
    b1, b2 = _tf2x32(key[0], key[1], np.zeros(2, np.uint32),
                     np.arange(2, dtype=np.uint32))
    out = np.empty((2, n), np.uint32)
    for i, sub in enumerate(((b1[0], b2[0]), (b1[1], b2[1]))):
        h, l = _tf2x32(sub[0], sub[1], np.zeros(n, np.uint32),
                       np.arange(n, dtype=np.uint32))
        out[i] = h ^ l
    span = maxval
    mult = (((2 ** 16 % span) * (2 ** 16 % span)) & 0xFFFFFFFF) % span
    off = ((out[0] % span) * np.uint64(mult) + out[1] % span) % span
    return off.astype(np.int32)


def _fold_in(key, data):
    o1, o2 = _tf2x32(key[0], key[1], np.zeros(1, np.uint32),
                     np.full(1, data, np.uint32))
    return (o1[0], o2[0])


def _build_index_tables():
    base = (np.uint32(0), np.uint32(42))
    idx1 = np.stack([_np_randint(_fold_in(base, 2 * b), _PAIRS, _N)
                     for b in range(_B)])
    idx2 = np.stack([_np_randint(_fold_in(base, 2 * b + 1), _PAIRS, _N)
                     for b in range(_B)])
    b_of_w = np.arange(_NW, dtype=np.int32) // _WPB
    chan = np.arange(_C, dtype=np.int32)

    def tables(idx):
        pixw = np.zeros((_NW, _PPW_PAD), np.int32)
        pixw[:, :_PPW] = idx.reshape(_NW, _PPW)
        flat = ((b_of_w[:, None, None] * _C + chan[None, :, None]) * _N
                + pixw[:, None, :])
        return np.ascontiguousarray(
            flat.reshape(_NW, _N_CHUNKS, _IDX_CHUNK).astype(np.int32))

    return tables(idx1), tables(idx2)


_F1, _F2 = _build_index_tables()


def _sqrt16(x):
    i = plsc.bitcast(x, jnp.int32)
    y = plsc.bitcast((i >> 1) + 0x1FBD1DF5, jnp.float32)
    for _ in range(3):
        y = 0.5 * (y + x / y)
    return y


def _sc_body(diff_hbm, ich_hbm, f1_hbm, f2_hbm, out_hbm,
             f1_v, f2_v, gi1, gi2, gd1, gd2, out_v, sem):
    wid = lax.axis_index("s") * 2 + lax.axis_index("c")

    pltpu.sync_copy(f1_hbm.at[wid], f1_v)
    pltpu.sync_copy(f2_hbm.at[wid], f2_v)

    copies = []
    for i in range(_N_CHUNKS):
        dst = pl.ds(i * _IDX_CHUNK, _IDX_CHUNK)
        copies.append(pltpu.async_copy(ich_hbm.at[f1_v.at[i]], gi1.at[dst], sem))
        copies.append(pltpu.async_copy(ich_hbm.at[f2_v.at[i]], gi2.at[dst], sem))
        copies.append(pltpu.async_copy(diff_hbm.at[f1_v.at[i]], gd1.at[dst], sem))
        copies.append(pltpu.async_copy(diff_hbm.at[f2_v.at[i]], gd2.at[dst], sem))
    for cp in copies:
        cp.wait()

    lane_iota = lax.iota(jnp.int32, 16)
    acc_s = jnp.zeros((16,), jnp.float32)
    acc_c = jnp.zeros((16,), jnp.float32)
    for j in range(_PPW_PAD // 16):
        d2_chro = jnp.zeros((16,), jnp.float32)
        d2_diff = jnp.zeros((16,), jnp.float32)
        for c in range(_C):
            sl = pl.ds(c * _PPW_PAD + j * 16, 16)
            d = gi1[sl] - gi2[sl]
            d2_chro = d2_chro + d * d
            e = gd1[sl] - gd2[sl]
            d2_diff = d2_diff + e * e
        same = d2_chro < 0.25
        valid = (j * 16 + lane_iota) < _PPW
        m = jnp.logical_and(same, valid)
        acc_s = acc_s + jnp.where(m, _sqrt16(d2_diff), 0.0)
        acc_c = acc_c + jnp.where(m, 1.0, 0.0)

    s = jnp.sum(acc_s)
    cnt = jnp.sum(acc_c)
    out_v[...] = jnp.where(lane_iota == 0, s,
                           jnp.where(lane_iota == 1, cnt, 0.0))
    pltpu.sync_copy(out_v, out_hbm.at[wid])


def _cluster_loss_impl(diffuse, Ichro):
    diff_flat = diffuse.reshape(_FLAT)
    ich_flat = Ichro.reshape(_FLAT)
    mesh = plsc.VectorSubcoreMesh(core_axis_name="c", subcore_axis_name="s",
                                  num_cores=2, num_subcores=16)
    fn = pl.kernel(
        _sc_body,
        out_type=jax.ShapeDtypeStruct((_NW, 16), jnp.float32),
        mesh=mesh,
        scratch_types=[
            pltpu.VMEM((_N_CHUNKS, _IDX_CHUNK), jnp.int32),
            pltpu.VMEM((_N_CHUNKS, _IDX_CHUNK), jnp.int32),
            pltpu.VMEM((_ELEMS_PER_SIDE,), jnp.float32),
            pltpu.VMEM((_ELEMS_PER_SIDE,), jnp.float32),
            pltpu.VMEM((_ELEMS_PER_SIDE,), jnp.float32),
            pltpu.VMEM((_ELEMS_PER_SIDE,), jnp.float32),
            pltpu.VMEM((16,), jnp.float32),
            pltpu.SemaphoreType.DMA,
        ],
        compiler_params=pltpu.CompilerParams(needs_layout_passes=False),
    )
    partials = fn(diff_flat, ich_flat, jnp.asarray(_F1), jnp.asarray(_F2))
    s_b = partials[:, 0].reshape(_B, _WPB).sum(axis=1)
    c_b = partials[:, 1].reshape(_B, _WPB).sum(axis=1)
    loss_b = jnp.where(c_b > 0, s_b / jnp.maximum(c_b, 1.0), 0.0)
    return (_LOSS_WEIGHT * jnp.sum(loss_b) / _B).astype(jnp.float32)


_cluster_loss = jax.jit(_cluster_loss_impl)


def kernel(diffuse, Ichro):
    return _cluster_loss(diffuse, Ichro)
